# baseline (speedup 1.0000x reference)
import jax
import jax.numpy as jnp
from jax.experimental import pallas as pl
from jax.experimental.pallas import tpu as pltpu
from jax.experimental.pallas import tpu_sc as plsc

_SHAPE = (2, 2, 3)
_N = 12


def _compact_body(t2_hbm, st_hbm, out_hbm, t2_s, st_s, out_s):
    pltpu.sync_copy(t2_hbm, t2_s)
    pltpu.sync_copy(st_hbm, st_s)

    st0 = st_s[0, 0, 0]
    for j in range(_N):
        out_s[j] = st0

    cnt = jnp.int32(0)
    for a in range(_SHAPE[0]):
        for b in range(_SHAPE[1]):
            for c in range(_SHAPE[2]):
                ok = t2_s[a, b, c] < 1.0

                @pl.when(ok)
                def _(a=a, b=b, c=c, cnt=cnt):
                    out_s[cnt] = st_s[a, b, c]

                cnt = cnt + jnp.where(ok, 1, 0)

    pltpu.sync_copy(out_s, out_hbm)


def kernel(t2, stored_tensor):
    mesh = plsc.ScalarSubcoreMesh(axis_name="c", num_cores=1)
    run = pl.kernel(
        _compact_body,
        mesh=mesh,
        out_type=jax.ShapeDtypeStruct((_N,), jnp.float32),
        scratch_types=[
            pltpu.SMEM(_SHAPE, jnp.float32),
            pltpu.SMEM(_SHAPE, jnp.float32),
            pltpu.SMEM((_N,), jnp.float32),
        ],
        compiler_params=pltpu.CompilerParams(
            needs_layout_passes=False,
            skip_device_barrier=True,
            disable_bounds_checks=True,
            disable_semaphore_checks=True,
        ),
    )
    return run(t2, stored_tensor)


# speedup vs baseline: 1.0006x; 1.0006x over previous
import jax
import jax.numpy as jnp
from jax.experimental import pallas as pl
from jax.experimental.pallas import tpu as pltpu
from jax.experimental.pallas import tpu_sc as plsc

_SHAPE = (2, 2, 3)
_N = 12


def _compact_body(t2_hbm, st_hbm, out_hbm, t2_s, st_s, out_s):
    pltpu.sync_copy(t2_hbm, t2_s)
    pltpu.sync_copy(st_hbm, st_s)

    st0 = st_s[0, 0, 0]
    for j in range(_N):
        out_s[j] = st0

    cnt = jnp.int32(0)
    for a in range(_SHAPE[0]):
        for b in range(_SHAPE[1]):
            for c in range(_SHAPE[2]):
                ok = t2_s[a, b, c] < 1.0

                @pl.when(ok)
                def _(a=a, b=b, c=c, cnt=cnt):
                    out_s[cnt] = st_s[a, b, c]

                cnt = cnt + jnp.where(ok, 1, 0)

    pltpu.sync_copy(out_s, out_hbm)


def kernel(t2, stored_tensor):
    mesh = plsc.ScalarSubcoreMesh(axis_name="c", num_cores=1)
    run = pl.kernel(
        _compact_body,
        mesh=mesh,
        out_type=jax.ShapeDtypeStruct((_N,), jnp.float32),
        scratch_types=[
            pltpu.SMEM(_SHAPE, jnp.float32),
            pltpu.SMEM(_SHAPE, jnp.float32),
            pltpu.SMEM((_N,), jnp.float32),
        ],
        compiler_params=pltpu.CompilerParams(needs_layout_passes=False),
    )
    return run(t2, stored_tensor)


# speedup vs baseline: 1.0380x; 1.0374x over previous
import jax
import jax.numpy as jnp
from jax import lax
from jax.experimental import pallas as pl
from jax.experimental.pallas import tpu as pltpu
from jax.experimental.pallas import tpu_sc as plsc

_SHAPE = (2, 2, 3)
_N = 12


def _compact_body(t2_hbm, st_hbm, out_hbm, t2_s, st_s, out_s, sem1, sem2):
    c1 = pltpu.make_async_copy(t2_hbm, t2_s, sem1)
    c2 = pltpu.make_async_copy(st_hbm, st_s, sem2)
    c1.start()
    c2.start()
    c1.wait()
    c2.wait()

    cnt = jnp.int32(0)
    for a in range(_SHAPE[0]):
        for b in range(_SHAPE[1]):
            for c in range(_SHAPE[2]):
                ok = t2_s[a, b, c] < 1.0

                @pl.when(ok)
                def _(a=a, b=b, c=c, cnt=cnt):
                    out_s[cnt] = st_s[a, b, c]

                cnt = cnt + jnp.where(ok, 1, 0)

    st0 = st_s[0, 0, 0]

    def pad_cond(j):
        return j < _N

    def pad_body(j):
        out_s[j] = st0
        return j + 1

    lax.while_loop(pad_cond, pad_body, cnt)

    pltpu.sync_copy(out_s, out_hbm)


def kernel(t2, stored_tensor):
    mesh = plsc.ScalarSubcoreMesh(axis_name="c", num_cores=1)
    run = pl.kernel(
        _compact_body,
        mesh=mesh,
        out_type=jax.ShapeDtypeStruct((_N,), jnp.float32),
        scratch_types=[
            pltpu.SMEM(_SHAPE, jnp.float32),
            pltpu.SMEM(_SHAPE, jnp.float32),
            pltpu.SMEM((_N,), jnp.float32),
            pltpu.SemaphoreType.DMA,
            pltpu.SemaphoreType.DMA,
        ],
        compiler_params=pltpu.CompilerParams(needs_layout_passes=False),
    )
    return run(t2, stored_tensor)


# speedup vs baseline: 1.0518x; 1.0134x over previous
import jax
import jax.numpy as jnp
from jax import lax
from jax.experimental import pallas as pl
from jax.experimental.pallas import tpu as pltpu
from jax.experimental.pallas import tpu_sc as plsc

_SHAPE = (2, 2, 3)
_N = 12


def _compact_body(t2_hbm, st_hbm, out_hbm, t2_s, st_s, out_s, sem1, sem2):
    c1 = pltpu.make_async_copy(t2_hbm, t2_s, sem1)
    c2 = pltpu.make_async_copy(st_hbm, st_s, sem2)
    c1.start()
    c2.start()
    c1.wait()

    cnt = jnp.int32(0)
    plan = []
    for a in range(_SHAPE[0]):
        for b in range(_SHAPE[1]):
            for c in range(_SHAPE[2]):
                ok = t2_s[a, b, c] < 1.0
                plan.append((a, b, c, ok, cnt))
                cnt = cnt + jnp.where(ok, 1, 0)

    c2.wait()
    for a, b, c, ok, pos in plan:

        @pl.when(ok)
        def _(a=a, b=b, c=c, pos=pos):
            out_s[pos] = st_s[a, b, c]

    st0 = st_s[0, 0, 0]

    def pad_cond(j):
        return j < _N

    def pad_body(j):
        out_s[j] = st0
        return j + 1

    lax.while_loop(pad_cond, pad_body, cnt)

    pltpu.sync_copy(out_s, out_hbm)


def kernel(t2, stored_tensor):
    mesh = plsc.ScalarSubcoreMesh(axis_name="c", num_cores=1)
    run = pl.kernel(
        _compact_body,
        mesh=mesh,
        out_type=jax.ShapeDtypeStruct((_N,), jnp.float32),
        scratch_types=[
            pltpu.SMEM(_SHAPE, jnp.float32),
            pltpu.SMEM(_SHAPE, jnp.float32),
            pltpu.SMEM((_N,), jnp.float32),
            pltpu.SemaphoreType.DMA,
            pltpu.SemaphoreType.DMA,
        ],
        compiler_params=pltpu.CompilerParams(needs_layout_passes=False),
    )
    return run(t2, stored_tensor)
